# baseline (speedup 1.0000x reference)
import functools

import jax
import jax.numpy as jnp
from jax import lax
from jax.experimental import pallas as pl
from jax.experimental.pallas import tpu as pltpu
from jax.experimental.pallas import tpu_sc as plsc

N = 100000
E = 3200000
G = 1024
F = 16

ER = E // 128
TILES = 32
CH = 2
NCH = 390
NPAIR = NCH // 2
AGGR_ROWS = 100096
ZR = 391

PPT = 3136
PPT_LAST = N - 31 * PPT
GP = 1040

_MESH = dict(core_axis_name="c", subcore_axis_name="s")
_SC_PARAMS = pltpu.CompilerParams(use_tc_tiling_on_sc=False)


def _encode(x, tid2, jW1, jb1, jW2, jb2, uW1, ub1, uW2, ub2):
    blk = 800
    grid = N // blk

    def body(x_ref, t_ref, jw1, jb1r, jw2, jb2r, uw1, ub1r, uw2, ub2r, o_ref):
        xb = x_ref[...]
        hj = jnp.maximum(xb @ jw1[...] + jb1r[...], 0.0) @ jw2[...] + jb2r[...]
        hu = jnp.maximum(xb @ uw1[...] + ub1r[...], 0.0) @ uw2[...] + ub2r[...]
        o_ref[...] = jnp.where(t_ref[...] == 0, hj, hu)

    w = lambda shp: pl.BlockSpec(shp, lambda i: (0,) * len(shp))
    return pl.pallas_call(
        body,
        grid=(grid,),
        in_specs=[
            pl.BlockSpec((blk, 5), lambda i: (i, 0)),
            pl.BlockSpec((blk, 1), lambda i: (i, 0)),
            w((5, F)), w((1, F)), w((F, F)), w((1, F)),
            w((5, F)), w((1, F)), w((F, F)), w((1, F)),
        ],
        out_specs=pl.BlockSpec((blk, F), lambda i: (i, 0)),
        out_shape=jax.ShapeDtypeStruct((N, F), jnp.float32),
    )(x, tid2, jW1, jb1, jW2, jb2, uW1, ub1, uW2, ub2)


def _edge_emb(ea32, W32_1, b1_128, W32_2, b2_128):
    blk = 1600
    grid = (E // 8) // blk

    def body(ea_ref, w1, b1, w2, b2, o1_ref, o2_ref):
        ea_b = ea_ref[...]
        o1_ref[...] = ea_b @ w1[...] + b1[...]
        o2_ref[...] = ea_b @ w2[...] + b2[...]

    w = lambda shp: pl.BlockSpec(shp, lambda i: (0,) * len(shp))
    return pl.pallas_call(
        body,
        grid=(grid,),
        in_specs=[
            pl.BlockSpec((blk, 32), lambda i: (i, 0)),
            w((32, 128)), w((1, 128)), w((32, 128)), w((1, 128)),
        ],
        out_specs=[
            pl.BlockSpec((blk, 128), lambda i: (i, 0)),
            pl.BlockSpec((blk, 128), lambda i: (i, 0)),
        ],
        out_shape=[
            jax.ShapeDtypeStruct((E // 8, 128), jnp.float32),
            jax.ShapeDtypeStruct((E // 8, 128), jnp.float32),
        ],
    )(ea32, W32_1, b1_128, W32_2, b2_128)


def _sc_gine(h, srcp, dstp, emb):
    mesh = plsc.VectorSubcoreMesh(**_MESH)

    @functools.partial(
        pl.kernel,
        mesh=mesh,
        compiler_params=_SC_PARAMS,
        out_type=jax.ShapeDtypeStruct((2, N, F), jnp.float32),
        scratch_types=[
            pltpu.VMEM((CH, 128), jnp.int32),
            pltpu.VMEM((CH, 128), jnp.int32),
            pltpu.VMEM((CH, 128), jnp.int32),
            pltpu.VMEM((CH, 128), jnp.int32),
            pltpu.VMEM((CH * 128 * F,), jnp.float32),
            pltpu.VMEM((CH * 128 * F,), jnp.float32),
            pltpu.VMEM((CH * 128, F), jnp.float32),
            pltpu.VMEM((CH * 128, F), jnp.float32),
            pltpu.VMEM((ZR, F), jnp.float32),
            pltpu.VMEM_SHARED((AGGR_ROWS, F), jnp.float32),
            pltpu.SemaphoreType.DMA,
            pltpu.SemaphoreType.DMA,
            pltpu.SemaphoreType.DMA,
            pltpu.SemaphoreType.DMA,
        ],
    )
    def k(h_hbm, src_hbm, dst_hbm, emb_hbm, out_hbm,
          sidx0, sidx1, didx0, didx1, emb0, emb1, rows0, rows1,
          zbuf, aggr, isem0, isem1, gsem0, gsem1):
        c = lax.axis_index("c")
        s = lax.axis_index("s")
        wid = s * 2 + c
        bufs = ((sidx0, didx0, emb0, rows0, isem0, gsem0),
                (sidx1, didx1, emb1, rows1, isem1, gsem1))

        @pl.loop(0, ZR)
        def _(i):
            zbuf[i] = jnp.zeros((F,), jnp.float32)

        zslice = AGGR_ROWS // 16
        for kk in range(zslice // ZR):
            pltpu.sync_copy(zbuf, aggr.at[pl.ds(s * zslice + kk * ZR, ZR)])
        plsc.subcore_barrier()

        rowbase = wid * 781 + jnp.minimum(wid, 8)
        nrows = jnp.where(wid < 8, 782, 781)

        def idx_copies(ck, b):
            sidx, didx, embv, _, isem, _ = bufs[b]
            rb = rowbase + ck * CH
            return (
                (src_hbm.at[pl.ds(rb, CH)], sidx, isem),
                (dst_hbm.at[pl.ds(rb, CH)], didx, isem),
                (emb_hbm.at[pl.ds(rb * (128 * F), CH * 128 * F)], embv, isem),
            )

        def issue_idx(ck, b):
            for tr in idx_copies(ck, b):
                pltpu.async_copy(*tr)

        def wait_idx(ck, b):
            for tr in idx_copies(ck, b):
                pltpu.make_async_copy(*tr).wait()

        def gather_copies(b):
            sidx, _, _, rows, _, gsem = bufs[b]
            return [
                (h_hbm.at[sidx.at[j]], rows.at[pl.ds(j * 128, 128)], gsem)
                for j in range(CH)
            ]

        def compute_scatter(b):
            _, didx, embv, rows, _, _ = bufs[b]

            @pl.loop(0, CH * 128)
            def _(e):
                rows[e] = jnp.maximum(rows[e] + embv[pl.ds(e * F, F)], 0.0)

            for j in range(CH):
                pltpu.sync_copy(rows.at[pl.ds(j * 128, 128)],
                                aggr.at[didx.at[j]], add=True)

        issue_idx(0, 0)
        wait_idx(0, 0)
        for tr in gather_copies(0):
            pltpu.async_copy(*tr)
        issue_idx(1, 1)

        def half(ci, k_off, b):
            ck = ci * 2 + k_off

            @pl.when(ck + 1 < NCH)
            def _():
                wait_idx(ck + 1, 1 - b)
                for tr in gather_copies(1 - b):
                    pltpu.async_copy(*tr)

            for tr in gather_copies(b):
                pltpu.make_async_copy(*tr).wait()
            compute_scatter(b)

            @pl.when(ck + 2 < NCH)
            def _():
                issue_idx(ck + 2, b)

        @pl.loop(0, NPAIR)
        def _(ci):
            half(ci, 0, 0)
            half(ci, 1, 1)

        @pl.loop(NCH * CH, nrows)
        def _(r):
            rb = rowbase + r
            pltpu.sync_copy(src_hbm.at[pl.ds(rb, 1)], sidx0.at[pl.ds(0, 1)])
            pltpu.sync_copy(dst_hbm.at[pl.ds(rb, 1)], didx0.at[pl.ds(0, 1)])
            pltpu.sync_copy(emb_hbm.at[pl.ds(rb * (128 * F), 128 * F)],
                            emb0.at[pl.ds(0, 128 * F)])
            pltpu.async_copy(h_hbm.at[sidx0.at[0]],
                             rows0.at[pl.ds(0, 128)], gsem0).wait()

            @pl.loop(0, 128)
            def _(e):
                rows0[e] = jnp.maximum(rows0[e] + emb0[pl.ds(e * F, F)], 0.0)

            pltpu.sync_copy(rows0.at[pl.ds(0, 128)],
                            aggr.at[didx0.at[0]], add=True)

        plsc.subcore_barrier()
        ob = s * 6256

        @pl.when(s < 15)
        def _():
            pltpu.sync_copy(aggr.at[pl.ds(ob, 6256)],
                            out_hbm.at[c, pl.ds(ob, 6256)])

        @pl.when(s == 15)
        def _():
            pltpu.sync_copy(aggr.at[pl.ds(ob, 6160)],
                            out_hbm.at[c, pl.ds(ob, 6160)])

    return k(h, srcp, dstp, emb)


def _mlp_ln(h, parts, W1, b1, W2, b2):
    blk = 800
    grid = N // blk

    def body(h_ref, p_ref, w1, b1r, w2, b2r, o_ref):
        t = h_ref[...] + p_ref[0] + p_ref[1]
        z = jnp.maximum(t @ w1[...] + b1r[...], 0.0) @ w2[...] + b2r[...]
        m = jnp.mean(z, axis=-1, keepdims=True)
        v = jnp.mean((z - m) * (z - m), axis=-1, keepdims=True)
        o_ref[...] = jnp.maximum((z - m) / jnp.sqrt(v + 1e-5), 0.0)

    w = lambda shp: pl.BlockSpec(shp, lambda i: (0,) * len(shp))
    return pl.pallas_call(
        body,
        grid=(grid,),
        in_specs=[
            pl.BlockSpec((blk, F), lambda i: (i, 0)),
            pl.BlockSpec((2, blk, F), lambda i: (0, i, 0)),
            w((F, 64)), w((1, 64)), w((64, F)), w((1, F)),
        ],
        out_specs=pl.BlockSpec((blk, F), lambda i: (i, 0)),
        out_shape=jax.ShapeDtypeStruct((N, F), jnp.float32),
    )(h, parts, W1, b1, W2, b2)


def _sc_pool(hp, batchp):
    mesh = plsc.VectorSubcoreMesh(**_MESH)

    @functools.partial(
        pl.kernel,
        mesh=mesh,
        compiler_params=_SC_PARAMS,
        out_type=(
            jax.ShapeDtypeStruct((TILES, GP, F), jnp.float32),
            jax.ShapeDtypeStruct((TILES, GP, F), jnp.float32),
            jax.ShapeDtypeStruct((TILES, GP, F), jnp.float32),
        ),
        scratch_types=[
            pltpu.VMEM((PPT, F), jnp.float32),
            pltpu.VMEM((PPT,), jnp.int32),
            pltpu.VMEM((GP, F), jnp.float32),
            pltpu.VMEM((GP, F), jnp.float32),
            pltpu.VMEM((GP, F), jnp.float32),
            pltpu.SemaphoreType.DMA,
        ],
    )
    def k(h_hbm, b_hbm, sum_o, max_o, cnt_o, rowsv, bv, sacc, macc, cacc, sem):
        c = lax.axis_index("c")
        s = lax.axis_index("s")
        wid = s * 2 + c
        base = wid * PPT

        e0 = jnp.where(lax.iota(jnp.int32, 16) == 0, 1.0, 0.0)

        @pl.loop(0, GP)
        def _(i):
            sacc[i] = jnp.zeros((F,), jnp.float32)
            macc[i] = jnp.full((F,), -jnp.inf, jnp.float32)
            cacc[i] = jnp.zeros((F,), jnp.float32)

        @pl.when(wid < 31)
        def _():
            cp1 = pltpu.async_copy(h_hbm.at[pl.ds(base, PPT)], rowsv, sem)
            cp2 = pltpu.async_copy(b_hbm.at[pl.ds(base, PPT)], bv, sem)
            cp1.wait()
            cp2.wait()

        @pl.when(wid == 31)
        def _():
            cp1 = pltpu.async_copy(h_hbm.at[pl.ds(31 * PPT, PPT_LAST)],
                                   rowsv.at[pl.ds(0, PPT_LAST)], sem)
            cp2 = pltpu.async_copy(b_hbm.at[pl.ds(31 * PPT, PPT_LAST)],
                                   bv.at[pl.ds(0, PPT_LAST)], sem)
            cp1.wait()
            cp2.wait()

        ngroups = jnp.where(wid == 31, PPT_LAST // 16, PPT // 16)

        @pl.loop(0, ngroups)
        def _(i):
            bvec = bv[pl.ds(i * 16, 16)]
            for kk in range(16):
                g = bvec[kk]
                row = rowsv[i * 16 + kk]
                sacc[g] = sacc[g] + row
                macc[g] = jnp.maximum(macc[g], row)
                cacc[g] = cacc[g] + e0

        pltpu.sync_copy(sacc, sum_o.at[wid])
        pltpu.sync_copy(macc, max_o.at[wid])
        pltpu.sync_copy(cacc, cnt_o.at[wid])

    return k(hp, batchp)


def _head(sums, maxs, cnts, fW1, fb1, fW2, fb2):
    def body(s_ref, m_ref, c_ref, w1, b1r, w2, b2r, o_ref):
        S = jnp.sum(s_ref[...], axis=0)[:G]
        M = jnp.max(m_ref[...], axis=0)[:G]
        Cn = jnp.sum(c_ref[...], axis=0)[:G, 0]
        mean = S / jnp.maximum(Cn, 1.0)[:, None]
        M = jnp.where(M > -jnp.inf, M, 0.0)
        g = jnp.concatenate([mean, M], axis=1)
        gm = jnp.mean(g, axis=-1, keepdims=True)
        gv = jnp.mean((g - gm) * (g - gm), axis=-1, keepdims=True)
        g = (g - gm) / jnp.sqrt(gv + 1e-5)
        o_ref[...] = jnp.maximum(g @ w1[...] + b1r[...], 0.0) @ w2[...] + b2r[...]

    return pl.pallas_call(
        body,
        out_shape=jax.ShapeDtypeStruct((G, 1), jnp.float32),
    )(sums, maxs, cnts, fW1, fb1, fW2, fb2)


def kernel(x, type_id, edge_index, edge_attr, batch,
           jW1, jb1, jW2, jb2, uW1, ub1, uW2, ub2,
           c1We, c1be, c1W1, c1b1, c1W2, c1b2,
           c2We, c2be, c2W1, c2b1, c2W2, c2b2,
           fW1, fb1, fW2, fb2):
    tid2 = type_id.reshape(N, 1)
    src2d = edge_index[0].reshape(ER, 128)
    dst2d = edge_index[1].reshape(ER, 128)
    ea32 = edge_attr.reshape(E // 8, 32)
    eye8 = jnp.eye(8, dtype=jnp.float32)
    emb1_2d, emb2_2d = _edge_emb(
        ea32,
        jnp.kron(eye8, c1We), jnp.tile(c1be, 8).reshape(1, 128),
        jnp.kron(eye8, c2We), jnp.tile(c2be, 8).reshape(1, 128))
    emb1 = emb1_2d.reshape(E * F)
    emb2 = emb2_2d.reshape(E * F)

    h0 = _encode(x, tid2,
                 jW1, jb1.reshape(1, F), jW2, jb2.reshape(1, F),
                 uW1, ub1.reshape(1, F), uW2, ub2.reshape(1, F))

    p1 = _sc_gine(h0, src2d, dst2d, emb1)
    h1 = _mlp_ln(h0, p1, c1W1, c1b1.reshape(1, 64), c1W2, c1b2.reshape(1, F))
    p2 = _sc_gine(h1, src2d, dst2d, emb2)
    h2 = _mlp_ln(h1, p2, c2W1, c2b1.reshape(1, 64), c2W2, c2b2.reshape(1, F))

    sums, maxs, cnts = _sc_pool(h2, batch)
    return _head(sums, maxs, cnts, fW1, fb1.reshape(1, 64),
                 fW2, fb2.reshape(1, 1))


# speedup vs baseline: 1.1509x; 1.1509x over previous
import functools

import jax
import jax.numpy as jnp
from jax import lax
from jax.experimental import pallas as pl
from jax.experimental.pallas import tpu as pltpu
from jax.experimental.pallas import tpu_sc as plsc

N = 100000
E = 3200000
G = 1024
F = 16

ER = E // 128
TILES = 32
CH = 2
NCH = 390
NPAIR = NCH // 2
AGGR_ROWS = 100096
ZR = 391

PPT = 3136
PPT_LAST = N - 31 * PPT
GP = 1040

_MESH = dict(core_axis_name="c", subcore_axis_name="s")
_SC_PARAMS = pltpu.CompilerParams(use_tc_tiling_on_sc=False)


def _encode(x, tid2, jW1, jb1, jW2, jb2, uW1, ub1, uW2, ub2):
    blk = 800
    grid = N // blk

    def body(x_ref, t_ref, jw1, jb1r, jw2, jb2r, uw1, ub1r, uw2, ub2r, o_ref):
        xb = x_ref[...]
        hj = jnp.maximum(xb @ jw1[...] + jb1r[...], 0.0) @ jw2[...] + jb2r[...]
        hu = jnp.maximum(xb @ uw1[...] + ub1r[...], 0.0) @ uw2[...] + ub2r[...]
        o_ref[...] = jnp.where(t_ref[...] == 0, hj, hu)

    w = lambda shp: pl.BlockSpec(shp, lambda i: (0,) * len(shp))
    return pl.pallas_call(
        body,
        grid=(grid,),
        in_specs=[
            pl.BlockSpec((blk, 5), lambda i: (i, 0)),
            pl.BlockSpec((blk, 1), lambda i: (i, 0)),
            w((5, F)), w((1, F)), w((F, F)), w((1, F)),
            w((5, F)), w((1, F)), w((F, F)), w((1, F)),
        ],
        out_specs=pl.BlockSpec((blk, F), lambda i: (i, 0)),
        out_shape=jax.ShapeDtypeStruct((N, F), jnp.float32),
    )(x, tid2, jW1, jb1, jW2, jb2, uW1, ub1, uW2, ub2)


def _edge_emb(ea32, W32, b128):
    blk = 1600
    grid = (E // 8) // blk

    def body(ea_ref, w1, b1, o_ref):
        o_ref[...] = ea_ref[...] @ w1[...] + b1[...]

    w = lambda shp: pl.BlockSpec(shp, lambda i: (0,) * len(shp))
    return pl.pallas_call(
        body,
        grid=(grid,),
        in_specs=[
            pl.BlockSpec((blk, 32), lambda i: (i, 0)),
            w((32, 128)), w((1, 128)),
        ],
        out_specs=pl.BlockSpec((blk, 128), lambda i: (i, 0)),
        out_shape=jax.ShapeDtypeStruct((E // 8, 128), jnp.float32),
    )(ea32, W32, b128)


def _edge_cast(edge_index):
    blk = 25600
    grid = E // blk

    def body(ei_ref, so_ref, do_ref):
        eb = ei_ref[...]
        so_ref[...] = eb[0].reshape(blk // 128, 128)
        do_ref[...] = eb[1].reshape(blk // 128, 128)

    return pl.pallas_call(
        body,
        grid=(grid,),
        in_specs=[pl.BlockSpec((2, blk), lambda i: (0, i))],
        out_specs=[
            pl.BlockSpec((blk // 128, 128), lambda i: (i, 0)),
            pl.BlockSpec((blk // 128, 128), lambda i: (i, 0)),
        ],
        out_shape=[
            jax.ShapeDtypeStruct((ER, 128), jnp.int32),
            jax.ShapeDtypeStruct((ER, 128), jnp.int32),
        ],
    )(edge_index)


def _sc_gine(h, srcp, dstp, emb):
    mesh = plsc.VectorSubcoreMesh(**_MESH)

    @functools.partial(
        pl.kernel,
        mesh=mesh,
        compiler_params=_SC_PARAMS,
        out_type=jax.ShapeDtypeStruct((2, N, F), jnp.float32),
        scratch_types=[
            pltpu.VMEM((CH, 128), jnp.int32),
            pltpu.VMEM((CH, 128), jnp.int32),
            pltpu.VMEM((CH, 128), jnp.int32),
            pltpu.VMEM((CH, 128), jnp.int32),
            pltpu.VMEM((CH * 16, 128), jnp.float32),
            pltpu.VMEM((CH * 16, 128), jnp.float32),
            pltpu.VMEM((CH * 128, F), jnp.float32),
            pltpu.VMEM((CH * 128, F), jnp.float32),
            pltpu.VMEM((ZR, F), jnp.float32),
            pltpu.VMEM_SHARED((AGGR_ROWS, F), jnp.float32),
            pltpu.SemaphoreType.DMA,
            pltpu.SemaphoreType.DMA,
            pltpu.SemaphoreType.DMA,
            pltpu.SemaphoreType.DMA,
        ],
    )
    def k(h_hbm, src_hbm, dst_hbm, emb_hbm, out_hbm,
          sidx0, sidx1, didx0, didx1, emb0, emb1, rows0, rows1,
          zbuf, aggr, isem0, isem1, gsem0, gsem1):
        c = lax.axis_index("c")
        s = lax.axis_index("s")
        wid = s * 2 + c
        bufs = ((sidx0, didx0, emb0, rows0, isem0, gsem0),
                (sidx1, didx1, emb1, rows1, isem1, gsem1))

        @pl.loop(0, ZR)
        def _(i):
            zbuf[i] = jnp.zeros((F,), jnp.float32)

        zslice = AGGR_ROWS // 16
        for kk in range(zslice // ZR):
            pltpu.sync_copy(zbuf, aggr.at[pl.ds(s * zslice + kk * ZR, ZR)])
        plsc.subcore_barrier()

        rowbase = wid * 781 + jnp.minimum(wid, 8)
        nrows = jnp.where(wid < 8, 782, 781)

        def idx_copies(ck, b):
            sidx, didx, embv, _, isem, _ = bufs[b]
            rb = rowbase + ck * CH
            return (
                (src_hbm.at[pl.ds(rb, CH)], sidx, isem),
                (dst_hbm.at[pl.ds(rb, CH)], didx, isem),
                (emb_hbm.at[pl.ds(rb * 16, CH * 16)], embv, isem),
            )

        def issue_idx(ck, b):
            for tr in idx_copies(ck, b):
                pltpu.async_copy(*tr)

        def wait_idx(ck, b):
            for tr in idx_copies(ck, b):
                pltpu.make_async_copy(*tr).wait()

        def gather_copies(b):
            sidx, _, _, rows, _, gsem = bufs[b]
            return [
                (h_hbm.at[sidx.at[j]], rows.at[pl.ds(j * 128, 128)], gsem)
                for j in range(CH)
            ]

        def compute_scatter(b):
            _, didx, embv, rows, _, _ = bufs[b]

            @pl.loop(0, CH * 16)
            def _(q):
                for j in range(8):
                    e = q * 8 + j
                    rows[e] = jnp.maximum(
                        rows[e] + embv[q, pl.ds(j * F, F)], 0.0)

            for j in range(CH):
                pltpu.sync_copy(rows.at[pl.ds(j * 128, 128)],
                                aggr.at[didx.at[j]], add=True)

        issue_idx(0, 0)
        wait_idx(0, 0)
        for tr in gather_copies(0):
            pltpu.async_copy(*tr)
        issue_idx(1, 1)

        def half(ci, k_off, b):
            ck = ci * 2 + k_off

            @pl.when(ck + 1 < NCH)
            def _():
                wait_idx(ck + 1, 1 - b)
                for tr in gather_copies(1 - b):
                    pltpu.async_copy(*tr)

            for tr in gather_copies(b):
                pltpu.make_async_copy(*tr).wait()
            compute_scatter(b)

            @pl.when(ck + 2 < NCH)
            def _():
                issue_idx(ck + 2, b)

        @pl.loop(0, NPAIR)
        def _(ci):
            half(ci, 0, 0)
            half(ci, 1, 1)

        @pl.loop(NCH * CH, nrows)
        def _(r):
            rb = rowbase + r
            pltpu.sync_copy(src_hbm.at[pl.ds(rb, 1)], sidx0.at[pl.ds(0, 1)])
            pltpu.sync_copy(dst_hbm.at[pl.ds(rb, 1)], didx0.at[pl.ds(0, 1)])
            pltpu.sync_copy(emb_hbm.at[pl.ds(rb * 16, 16)],
                            emb0.at[pl.ds(0, 16)])
            pltpu.async_copy(h_hbm.at[sidx0.at[0]],
                             rows0.at[pl.ds(0, 128)], gsem0).wait()

            @pl.loop(0, 16)
            def _(q):
                for j in range(8):
                    e = q * 8 + j
                    rows0[e] = jnp.maximum(
                        rows0[e] + emb0[q, pl.ds(j * F, F)], 0.0)

            pltpu.sync_copy(rows0.at[pl.ds(0, 128)],
                            aggr.at[didx0.at[0]], add=True)

        plsc.subcore_barrier()
        ob = s * 6256

        @pl.when(s < 15)
        def _():
            pltpu.sync_copy(aggr.at[pl.ds(ob, 6256)],
                            out_hbm.at[c, pl.ds(ob, 6256)])

        @pl.when(s == 15)
        def _():
            pltpu.sync_copy(aggr.at[pl.ds(ob, 6160)],
                            out_hbm.at[c, pl.ds(ob, 6160)])

    return k(h, srcp, dstp, emb)


def _mlp_ln(h, parts, W1, b1, W2, b2):
    blk = 800
    grid = N // blk

    def body(h_ref, p_ref, w1, b1r, w2, b2r, o_ref):
        t = h_ref[...] + p_ref[0] + p_ref[1]
        z = jnp.maximum(t @ w1[...] + b1r[...], 0.0) @ w2[...] + b2r[...]
        m = jnp.mean(z, axis=-1, keepdims=True)
        v = jnp.mean((z - m) * (z - m), axis=-1, keepdims=True)
        o_ref[...] = jnp.maximum((z - m) / jnp.sqrt(v + 1e-5), 0.0)

    w = lambda shp: pl.BlockSpec(shp, lambda i: (0,) * len(shp))
    return pl.pallas_call(
        body,
        grid=(grid,),
        in_specs=[
            pl.BlockSpec((blk, F), lambda i: (i, 0)),
            pl.BlockSpec((2, blk, F), lambda i: (0, i, 0)),
            w((F, 64)), w((1, 64)), w((64, F)), w((1, F)),
        ],
        out_specs=pl.BlockSpec((blk, F), lambda i: (i, 0)),
        out_shape=jax.ShapeDtypeStruct((N, F), jnp.float32),
    )(h, parts, W1, b1, W2, b2)


def _sc_pool(hp, batchp):
    mesh = plsc.VectorSubcoreMesh(**_MESH)

    @functools.partial(
        pl.kernel,
        mesh=mesh,
        compiler_params=_SC_PARAMS,
        out_type=(
            jax.ShapeDtypeStruct((TILES, GP, F), jnp.float32),
            jax.ShapeDtypeStruct((TILES, GP, F), jnp.float32),
            jax.ShapeDtypeStruct((TILES, GP, F), jnp.float32),
        ),
        scratch_types=[
            pltpu.VMEM((PPT, F), jnp.float32),
            pltpu.VMEM((PPT,), jnp.int32),
            pltpu.VMEM((GP, F), jnp.float32),
            pltpu.VMEM((GP, F), jnp.float32),
            pltpu.VMEM((GP, F), jnp.float32),
            pltpu.SemaphoreType.DMA,
        ],
    )
    def k(h_hbm, b_hbm, sum_o, max_o, cnt_o, rowsv, bv, sacc, macc, cacc, sem):
        c = lax.axis_index("c")
        s = lax.axis_index("s")
        wid = s * 2 + c
        base = wid * PPT

        e0 = jnp.where(lax.iota(jnp.int32, 16) == 0, 1.0, 0.0)

        @pl.loop(0, GP)
        def _(i):
            sacc[i] = jnp.zeros((F,), jnp.float32)
            macc[i] = jnp.full((F,), -jnp.inf, jnp.float32)
            cacc[i] = jnp.zeros((F,), jnp.float32)

        @pl.when(wid < 31)
        def _():
            cp1 = pltpu.async_copy(h_hbm.at[pl.ds(base, PPT)], rowsv, sem)
            cp2 = pltpu.async_copy(b_hbm.at[pl.ds(base, PPT)], bv, sem)
            cp1.wait()
            cp2.wait()

        @pl.when(wid == 31)
        def _():
            cp1 = pltpu.async_copy(h_hbm.at[pl.ds(31 * PPT, PPT_LAST)],
                                   rowsv.at[pl.ds(0, PPT_LAST)], sem)
            cp2 = pltpu.async_copy(b_hbm.at[pl.ds(31 * PPT, PPT_LAST)],
                                   bv.at[pl.ds(0, PPT_LAST)], sem)
            cp1.wait()
            cp2.wait()

        ngroups = jnp.where(wid == 31, PPT_LAST // 16, PPT // 16)

        @pl.loop(0, ngroups)
        def _(i):
            bvec = bv[pl.ds(i * 16, 16)]
            for kk in range(16):
                g = bvec[kk]
                row = rowsv[i * 16 + kk]
                sacc[g] = sacc[g] + row
                macc[g] = jnp.maximum(macc[g], row)
                cacc[g] = cacc[g] + e0

        pltpu.sync_copy(sacc, sum_o.at[wid])
        pltpu.sync_copy(macc, max_o.at[wid])
        pltpu.sync_copy(cacc, cnt_o.at[wid])

    return k(hp, batchp)


def _head(sums, maxs, cnts, fW1, fb1, fW2, fb2):
    def body(s_ref, m_ref, c_ref, w1, b1r, w2, b2r, o_ref):
        S = jnp.sum(s_ref[...], axis=0)[:G]
        M = jnp.max(m_ref[...], axis=0)[:G]
        Cn = jnp.sum(c_ref[...], axis=0)[:G, 0]
        mean = S / jnp.maximum(Cn, 1.0)[:, None]
        M = jnp.where(M > -jnp.inf, M, 0.0)
        g = jnp.concatenate([mean, M], axis=1)
        gm = jnp.mean(g, axis=-1, keepdims=True)
        gv = jnp.mean((g - gm) * (g - gm), axis=-1, keepdims=True)
        g = (g - gm) / jnp.sqrt(gv + 1e-5)
        o_ref[...] = jnp.maximum(g @ w1[...] + b1r[...], 0.0) @ w2[...] + b2r[...]

    return pl.pallas_call(
        body,
        out_shape=jax.ShapeDtypeStruct((G, 1), jnp.float32),
    )(sums, maxs, cnts, fW1, fb1, fW2, fb2)


def kernel(x, type_id, edge_index, edge_attr, batch,
           jW1, jb1, jW2, jb2, uW1, ub1, uW2, ub2,
           c1We, c1be, c1W1, c1b1, c1W2, c1b2,
           c2We, c2be, c2W1, c2b1, c2W2, c2b2,
           fW1, fb1, fW2, fb2):
    tid2 = type_id.reshape(N, 1)
    src2d, dst2d = _edge_cast(edge_index)
    ea32 = edge_attr.reshape(E // 8, 32)
    eye8 = jnp.eye(8, dtype=jnp.float32)
    emb1 = _edge_emb(ea32, jnp.kron(eye8, c1We),
                     jnp.tile(c1be, 8).reshape(1, 128))
    emb2 = _edge_emb(ea32, jnp.kron(eye8, c2We),
                     jnp.tile(c2be, 8).reshape(1, 128))

    h0 = _encode(x, tid2,
                 jW1, jb1.reshape(1, F), jW2, jb2.reshape(1, F),
                 uW1, ub1.reshape(1, F), uW2, ub2.reshape(1, F))

    p1 = _sc_gine(h0, src2d, dst2d, emb1)
    h1 = _mlp_ln(h0, p1, c1W1, c1b1.reshape(1, 64), c1W2, c1b2.reshape(1, F))
    p2 = _sc_gine(h1, src2d, dst2d, emb2)
    h2 = _mlp_ln(h1, p2, c2W1, c2b1.reshape(1, 64), c2W2, c2b2.reshape(1, F))

    sums, maxs, cnts = _sc_pool(h2, batch)
    return _head(sums, maxs, cnts, fW1, fb1.reshape(1, 64),
                 fW2, fb2.reshape(1, 1))
